# Initial kernel scaffold; baseline (speedup 1.0000x reference)
import jax
import jax.numpy as jnp
from jax.experimental import pallas as pl


def kernel(coords, images):
    raise NotImplementedError("write your pallas kernel here")


# speedup vs baseline: 1.3447x; 1.3447x over previous
import functools

import jax
import jax.numpy as jnp
from jax import lax
from jax.experimental import pallas as pl
from jax.experimental.pallas import tpu as pltpu
from jax.experimental.pallas import tpu_sc as plsc

_NC, _NS, _L = 2, 16, 16
_NW = _NC * _NS
_NBINS = 256
_PAD = 16
_HIST = _NBINS + 2 * _PAD
_R = 8
_PPS = 14
_NHS = 224
_HSW = _NHS // _NW
_NP = 3136
_C = 0.5 / (2.55 * 2.55)


def _sc_hist(depth_hs):
    mesh = plsc.VectorSubcoreMesh(core_axis_name="c", subcore_axis_name="s")

    @functools.partial(
        pl.kernel,
        mesh=mesh,
        compiler_params=pltpu.CompilerParams(use_tc_tiling_on_sc=False,
                                             needs_layout_passes=False),
        out_type=jax.ShapeDtypeStruct((_NP, _NBINS), jnp.float32),
        scratch_types=[
            pltpu.VMEM((_R, _PPS * _R), jnp.float32),
            pltpu.VMEM((_HSW, _PPS, _NBINS), jnp.float32),
            pltpu.VMEM((_L * _HIST,), jnp.float32),
            pltpu.SemaphoreType.DMA,
        ],
    )
    def k(depth_hbm, out_hbm, in_v, stage_v, hist_v, sem):
        wid = lax.axis_index("s") * _NC + lax.axis_index("c")
        iota = lax.iota(jnp.int32, _L)
        lanebase = iota * _HIST
        zeros = jnp.zeros((_L,), jnp.float32)
        colv = [jnp.minimum(iota * _R + cc, _PPS * _R - 1) for cc in range(_R)]

        def zero_body(i, carry):
            hist_v[pl.ds(_L * i, _L)] = zeros
            return carry

        lax.fori_loop(0, _HIST, zero_body, 0)

        def strip_body(t, carry):
            hs = wid * _HSW + t
            pltpu.sync_copy(depth_hbm.at[hs], in_v)

            def row_body(r, carry2):
                rowv = jnp.broadcast_to(r, (_L,))
                for cc in range(_R):
                    pix = plsc.load_gather(in_v, [rowv, colv[cc]])
                    u = pix * 255.0
                    j0 = u.astype(jnp.int32)
                    frac = u - j0.astype(jnp.float32)
                    sidx = lanebase + j0
                    for kk in range(_L):
                        d = frac + float(8 - kk)
                        v = jnp.exp(d * d * (-_C))
                        plsc.addupdate_scatter(hist_v, [sidx + (kk + 8)], v)
                return carry2

            lax.fori_loop(0, _R, row_body, 0)

            def readout_body(p, carry3):
                for i in range(_NBINS // _L):
                    src = pl.ds(p * _HIST + _PAD + _L * i, _L)
                    vals = hist_v[src]
                    hist_v[src] = zeros
                    stage_v[t, p, pl.ds(_L * i, _L)] = vals
                return carry3

            lax.fori_loop(0, _PPS, readout_body, 0)
            pltpu.async_copy(stage_v.at[t],
                             out_hbm.at[pl.ds(hs * _PPS, _PPS)], sem)
            return carry

        lax.fori_loop(0, _HSW, strip_body, 0)
        for t in range(_HSW):
            pltpu.make_async_copy(
                stage_v.at[t],
                out_hbm.at[pl.ds(wid * _HSW * _PPS + t * _PPS, _PPS)],
                sem).wait()

    return k(depth_hs)


def _tc_entropy(hist):
    def body(h_ref, o_ref):
        s = h_ref[...] * (1.0 / 64.0)
        tot = jnp.sum(s, axis=1, keepdims=True) + 1e-10
        pdf = s / tot
        o_ref[...] = -jnp.sum(pdf * jnp.log(pdf + 1e-10), axis=1,
                              keepdims=True)

    return pl.pallas_call(
        body,
        out_shape=jax.ShapeDtypeStruct((_NP, 1), jnp.float32),
    )(hist)


def kernel(coords, images):
    del coords
    depth = images[:, :, -1]
    d5 = depth.reshape(4, 28, _R, 2, _PPS * _R)
    depth_hs = d5.transpose(0, 1, 3, 2, 4).reshape(_NHS, _R, _PPS * _R)
    hist = _sc_hist(depth_hs)
    ent = _tc_entropy(hist)
    return ent.reshape(2, 2, 1, 28, 28)
